# baseline (speedup 1.0000x reference)
import functools
import math

import jax
import jax.numpy as jnp
from jax import lax
from jax.experimental import pallas as pl
from jax.experimental.pallas import tpu as pltpu
from jax.experimental.pallas import tpu_sc as plsc

D = 64
LANES = 16
NC, NS = 2, 16
NW = NC * NS
CHUNK = 256
SUB = 128
KSUB = CHUNK // SUB
NBUF = 2


@functools.lru_cache(maxsize=None)
def _build(B, V):
    assert B % (NW * CHUNK) == 0
    b_per_w = B // NW
    n_chunks = b_per_w // CHUNK
    n_groups = n_chunks // NBUF
    idx_rows_w = b_per_w // SUB
    scale = jnp.float32(math.sqrt(D))

    mesh = plsc.VectorSubcoreMesh(core_axis_name="c", subcore_axis_name="s")

    @functools.partial(
        pl.kernel,
        out_type=jax.ShapeDtypeStruct((B, D), jnp.float32),
        mesh=mesh,
        compiler_params=pltpu.CompilerParams(use_tc_tiling_on_sc=False),
        scratch_types=[
            pltpu.VMEM((idx_rows_w, SUB), jnp.int32),
            pltpu.VMEM((NBUF, CHUNK, D), jnp.float32),
            pltpu.VMEM((NBUF, CHUNK, D), jnp.float32),
            pltpu.SemaphoreType.DMA((NBUF,)),
            pltpu.SemaphoreType.DMA((NBUF,)),
        ],
    )
    def emb(w_hbm, x_hbm, out_hbm, idx_v, rows_in, rows_out, gsem, osem):
        wid = lax.axis_index("s") * NC + lax.axis_index("c")
        base = wid * b_per_w

        pltpu.sync_copy(x_hbm.at[pl.ds(wid * idx_rows_w, idx_rows_w)], idx_v)

        def fire_gather(g, b):
            for j in range(KSUB):
                pltpu.async_copy(
                    w_hbm.at[idx_v.at[g * KSUB + j]],
                    rows_in.at[b, pl.ds(j * SUB, SUB)],
                    gsem.at[b],
                )

        def wait_gather(b):
            for j in range(KSUB):
                pltpu.make_async_copy(
                    w_hbm.at[idx_v.at[j]],
                    rows_in.at[b, pl.ds(j * SUB, SUB)],
                    gsem.at[b],
                ).wait()

        def fire_scatter(g, b):
            pltpu.async_copy(
                rows_out.at[b],
                out_hbm.at[pl.ds(base + g * CHUNK, CHUNK)],
                osem.at[b],
            )

        def wait_scatter(b):
            pltpu.make_async_copy(
                rows_out.at[b],
                out_hbm.at[pl.ds(base, CHUNK)],
                osem.at[b],
            ).wait()

        def scale_chunk(b):
            def body(i, _):
                for r in range(4):
                    for j in range(D // LANES):
                        s = pl.ds(j * LANES, LANES)
                        rows_out[b, i * 4 + r, s] = rows_in[b, i * 4 + r, s] * scale
                return 0

            lax.fori_loop(0, CHUNK // 4, body, 0)

        for b in range(NBUF):
            fire_gather(jnp.int32(b), b)

        def group(t, _):
            for b in range(NBUF):
                g = t * NBUF + b
                wait_gather(b)

                @pl.when(t > 0)
                def _():
                    wait_scatter(b)

                scale_chunk(b)
                fire_scatter(g, b)

                @pl.when(t < n_groups - 1)
                def _():
                    fire_gather(g + NBUF, b)

            return 0

        lax.fori_loop(0, n_groups, group, 0)

        for b in range(NBUF):
            wait_scatter(b)

    return emb


def kernel(x, W):
    Bt, H = x.shape
    B = Bt * H
    V, d = W.shape
    xf = x.reshape(B // SUB, SUB).astype(jnp.int32)
    out = _build(B, V)(W, xf)
    return out.reshape(Bt, H, d)
